# baseline (speedup 1.0000x reference)
import jax
import jax.numpy as jnp
from jax.experimental import pallas as pl
from jax.experimental.pallas import tpu as pltpu

_B, _C, _H, _W = 128, 3, 256, 256
_CHW = _C * _H * _W
_LANES = 128
_SUB = _CHW // _LANES
_R = 8


def _qsample_body(t_ref, sac_ref, som_ref, x_ref, n_ref, o_ref):
    i = pl.program_id(0)
    for r in range(_R):
        tt = t_ref[i * _R + r]
        a = sac_ref[tt]
        b = som_ref[tt]
        o_ref[r] = a * x_ref[r] + b * n_ref[r]


def kernel(x_start, t, noise, sqrt_alphas_cumprod, sqrt_one_minus_alphas_cumprod):
    grid_spec = pltpu.PrefetchScalarGridSpec(
        num_scalar_prefetch=3,
        grid=(_B // _R,),
        in_specs=[
            pl.BlockSpec((_R, _C, _H, _W), lambda i, *_: (i, 0, 0, 0)),
            pl.BlockSpec((_R, _C, _H, _W), lambda i, *_: (i, 0, 0, 0)),
        ],
        out_specs=pl.BlockSpec((_R, _C, _H, _W), lambda i, *_: (i, 0, 0, 0)),
    )

    return pl.pallas_call(
        _qsample_body,
        grid_spec=grid_spec,
        out_shape=jax.ShapeDtypeStruct((_B, _C, _H, _W), jnp.float32),
        compiler_params=pltpu.CompilerParams(dimension_semantics=("parallel",)),
    )(t, sqrt_alphas_cumprod, sqrt_one_minus_alphas_cumprod, x_start, noise)


# speedup vs baseline: 1.0038x; 1.0038x over previous
import jax
import jax.numpy as jnp
from jax.experimental import pallas as pl
from jax.experimental.pallas import tpu as pltpu

_B, _C, _H, _W = 128, 3, 256, 256
_R = 4


def _qsample_body(t_ref, sac_ref, som_ref, x_ref, n_ref, o_ref):
    i = pl.program_id(0)
    for r in range(_R):
        tt = t_ref[i * _R + r]
        a = sac_ref[tt]
        b = som_ref[tt]
        o_ref[r] = a * x_ref[r] + b * n_ref[r]


def kernel(x_start, t, noise, sqrt_alphas_cumprod, sqrt_one_minus_alphas_cumprod):
    grid_spec = pltpu.PrefetchScalarGridSpec(
        num_scalar_prefetch=3,
        grid=(_B // _R,),
        in_specs=[
            pl.BlockSpec((_R, _C, _H, _W), lambda i, *_: (i, 0, 0, 0)),
            pl.BlockSpec((_R, _C, _H, _W), lambda i, *_: (i, 0, 0, 0)),
        ],
        out_specs=pl.BlockSpec((_R, _C, _H, _W), lambda i, *_: (i, 0, 0, 0)),
    )

    return pl.pallas_call(
        _qsample_body,
        grid_spec=grid_spec,
        out_shape=jax.ShapeDtypeStruct((_B, _C, _H, _W), jnp.float32),
        compiler_params=pltpu.CompilerParams(dimension_semantics=("parallel",)),
    )(t, sqrt_alphas_cumprod, sqrt_one_minus_alphas_cumprod, x_start, noise)
